# Initial kernel scaffold; baseline (speedup 1.0000x reference)
import functools

import jax
import jax.numpy as jnp
from jax import lax
from jax.experimental import pallas as pl
from jax.experimental.pallas import tpu as pltpu
from jax.experimental.pallas import tpu_sc as plsc

N = 10000
E = 320000
D = 128

NC = 2
NT = 16
EC = E // NC
ET = EC // NT
G = 80
NCH = ET // G
RT = N // NT
ZR = 125

_MESH = plsc.VectorSubcoreMesh(
    core_axis_name="c", subcore_axis_name="s", num_cores=NC, num_subcores=NT
)

ROWBLK = 1000


@functools.partial(
    pl.kernel,
    out_type=jax.ShapeDtypeStruct((NC, 2, N, 16), jnp.float32),
    mesh=_MESH,
    scratch_types=[
        pltpu.VMEM((G,), jnp.int32),
        pltpu.VMEM((G, 16), jnp.float32),
        pltpu.VMEM((ZR, 16), jnp.float32),
        pltpu.VMEM_SHARED((N, 16), jnp.float32),
        pltpu.VMEM_SHARED((N, 16), jnp.float32),
    ],
)
def _sc_degrees(src_hbm, dst_hbm, out_hbm, idx_v, ones_v, zbuf, hsrc_sh, hdst_sh):
    cid = lax.axis_index("c")
    sid = lax.axis_index("s")

    @pl.loop(0, G)
    def _(i):
        ones_v[i, :] = jnp.ones((16,), jnp.float32)

    @pl.loop(0, ZR)
    def _(i):
        zbuf[i, :] = jnp.zeros((16,), jnp.float32)

    @pl.loop(0, RT // ZR)
    def _(j):
        pltpu.sync_copy(zbuf, hsrc_sh.at[pl.ds(sid * RT + j * ZR, ZR)])
        pltpu.sync_copy(zbuf, hdst_sh.at[pl.ds(sid * RT + j * ZR, ZR)])

    plsc.subcore_barrier()

    base = cid * EC + sid * ET

    @pl.loop(0, NCH)
    def _(ch):
        pltpu.sync_copy(src_hbm.at[pl.ds(base + ch * G, G)], idx_v)
        pltpu.sync_copy(ones_v, hsrc_sh.at[idx_v], add=True)

    @pl.loop(0, NCH)
    def _(ch):
        pltpu.sync_copy(dst_hbm.at[pl.ds(base + ch * G, G)], idx_v)
        pltpu.sync_copy(ones_v, hdst_sh.at[idx_v], add=True)

    plsc.subcore_barrier()

    pltpu.sync_copy(hsrc_sh.at[pl.ds(sid * RT, RT)],
                    out_hbm.at[cid, 0, pl.ds(sid * RT, RT)])
    pltpu.sync_copy(hdst_sh.at[pl.ds(sid * RT, RT)],
                    out_hbm.at[cid, 1, pl.ds(sid * RT, RT)])


@functools.partial(
    pl.kernel,
    out_type=jax.ShapeDtypeStruct((NC, N, D), jnp.float32),
    mesh=_MESH,
    scratch_types=[
        pltpu.VMEM((G,), jnp.int32),
        pltpu.VMEM((G,), jnp.int32),
        pltpu.VMEM((G, D), jnp.float32),
        pltpu.VMEM((ZR, D), jnp.float32),
        pltpu.VMEM_SHARED((N, D), jnp.float32),
        pltpu.SemaphoreType.DMA,
    ],
)
def _sc_segsum(hw_hbm, src_hbm, dst_hbm, out_hbm,
               idx_s, idx_d, rows_v, zbuf, agg_sh, sem):
    cid = lax.axis_index("c")
    sid = lax.axis_index("s")

    @pl.loop(0, ZR)
    def _(i):
        @pl.loop(0, D, step=16)
        def _(j):
            zbuf[i, pl.ds(j, 16)] = jnp.zeros((16,), jnp.float32)

    @pl.loop(0, RT // ZR)
    def _(j):
        pltpu.sync_copy(zbuf, agg_sh.at[pl.ds(sid * RT + j * ZR, ZR)])

    plsc.subcore_barrier()

    base = cid * EC + sid * ET

    @pl.loop(0, NCH)
    def _(ch):
        pltpu.sync_copy(src_hbm.at[pl.ds(base + ch * G, G)], idx_s)
        pltpu.sync_copy(dst_hbm.at[pl.ds(base + ch * G, G)], idx_d)
        pltpu.async_copy(hw_hbm.at[idx_s], rows_v, sem).wait()
        pltpu.sync_copy(rows_v, agg_sh.at[idx_d], add=True)

    plsc.subcore_barrier()

    pltpu.sync_copy(agg_sh.at[pl.ds(sid * RT, RT)],
                    out_hbm.at[cid, pl.ds(sid * RT, RT)])


@functools.partial(
    pl.kernel,
    out_type=[
        jax.ShapeDtypeStruct((E,), jnp.float32),
        jax.ShapeDtypeStruct((E,), jnp.float32),
    ],
    mesh=_MESH,
    scratch_types=[
        pltpu.VMEM((G,), jnp.int32),
        pltpu.VMEM((G,), jnp.int32),
        pltpu.VMEM((G, D), jnp.float32),
        pltpu.VMEM((G, D), jnp.float32),
        pltpu.VMEM((ET,), jnp.float32),
        pltpu.SemaphoreType.DMA,
        pltpu.SemaphoreType.DMA,
    ],
)
def _sc_scores(h_hbm, psrc, pdst, nsrc, ndst, pos_out, neg_out,
               idx_s, idx_d, hs, hd, score_v, sem1, sem2):
    cid = lax.axis_index("c")
    sid = lax.axis_index("s")
    base = cid * EC + sid * ET

    def one_set(src_arr, dst_arr, out_arr):
        @pl.loop(0, NCH)
        def _(ch):
            pltpu.sync_copy(src_arr.at[pl.ds(base + ch * G, G)], idx_s)
            pltpu.sync_copy(dst_arr.at[pl.ds(base + ch * G, G)], idx_d)
            cp1 = pltpu.async_copy(h_hbm.at[idx_s], hs, sem1)
            cp2 = pltpu.async_copy(h_hbm.at[idx_d], hd, sem2)
            cp1.wait()
            cp2.wait()

            @pl.loop(0, G)
            def _(e):
                acc = hs[e, pl.ds(0, 16)] * hd[e, pl.ds(0, 16)]
                for j in range(1, D // 16):
                    acc = acc + hs[e, pl.ds(j * 16, 16)] * hd[e, pl.ds(j * 16, 16)]
                score_v[ch * G + e] = jnp.sum(acc)

        pltpu.sync_copy(score_v, out_arr.at[pl.ds(base, ET)])

    one_set(psrc, pdst, pos_out)
    one_set(nsrc, ndst, neg_out)


def _tc1_body(deg_ref, x_ref, w_ref, out_ref):
    dout = deg_ref[0, 0] + deg_ref[1, 0]
    ns = lax.rsqrt(jnp.maximum(dout[:, 0:1], 1.0))
    xw = jnp.dot(x_ref[...], w_ref[...], preferred_element_type=jnp.float32)
    out_ref[...] = xw * ns


def _tc_linear1(degp, x, W1):
    return pl.pallas_call(
        _tc1_body,
        grid=(N // ROWBLK,),
        in_specs=[
            pl.BlockSpec((NC, 2, ROWBLK, 16), lambda i: (0, 0, i, 0)),
            pl.BlockSpec((ROWBLK, D), lambda i: (i, 0)),
            pl.BlockSpec((D, D), lambda i: (0, 0)),
        ],
        out_specs=pl.BlockSpec((ROWBLK, D), lambda i: (i, 0)),
        out_shape=jax.ShapeDtypeStruct((N, D), jnp.float32),
    )(degp, x, W1)


def _tc2_body(deg_ref, s_ref, b_ref, w_ref, out_ref):
    din = deg_ref[0, 1] + deg_ref[1, 1]
    nd = lax.rsqrt(jnp.maximum(din[:, 0:1], 1.0))
    dout = deg_ref[0, 0] + deg_ref[1, 0]
    ns = lax.rsqrt(jnp.maximum(dout[:, 0:1], 1.0))
    h1 = jnp.maximum((s_ref[0] + s_ref[1]) * nd + b_ref[...], 0.0)
    hw = jnp.dot(h1, w_ref[...], preferred_element_type=jnp.float32)
    out_ref[...] = hw * ns


def _tc_mid(degp, s1, b1, W2):
    return pl.pallas_call(
        _tc2_body,
        grid=(N // ROWBLK,),
        in_specs=[
            pl.BlockSpec((NC, 2, ROWBLK, 16), lambda i: (0, 0, i, 0)),
            pl.BlockSpec((NC, ROWBLK, D), lambda i: (0, i, 0)),
            pl.BlockSpec((1, D), lambda i: (0, 0)),
            pl.BlockSpec((D, D), lambda i: (0, 0)),
        ],
        out_specs=pl.BlockSpec((ROWBLK, D), lambda i: (i, 0)),
        out_shape=jax.ShapeDtypeStruct((N, D), jnp.float32),
    )(degp, s1, b1.reshape(1, D), W2)


def _tc3_body(deg_ref, s_ref, b_ref, out_ref):
    din = deg_ref[0, 1] + deg_ref[1, 1]
    nd = lax.rsqrt(jnp.maximum(din[:, 0:1], 1.0))
    out_ref[...] = (s_ref[0] + s_ref[1]) * nd + b_ref[...]


def _tc_final(degp, s2, b2):
    return pl.pallas_call(
        _tc3_body,
        grid=(N // ROWBLK,),
        in_specs=[
            pl.BlockSpec((NC, 2, ROWBLK, 16), lambda i: (0, 0, i, 0)),
            pl.BlockSpec((NC, ROWBLK, D), lambda i: (0, i, 0)),
            pl.BlockSpec((1, D), lambda i: (0, 0)),
        ],
        out_specs=pl.BlockSpec((ROWBLK, D), lambda i: (i, 0)),
        out_shape=jax.ShapeDtypeStruct((N, D), jnp.float32),
    )(degp, s2, b2.reshape(1, D))


def kernel(x, pos_edge_index, neg_edge_index, W1, b1, W2, b2):
    psrc = pos_edge_index[0].astype(jnp.int32)
    pdst = pos_edge_index[1].astype(jnp.int32)
    nsrc = neg_edge_index[0].astype(jnp.int32)
    ndst = neg_edge_index[1].astype(jnp.int32)

    degp = _sc_degrees(psrc, pdst)
    hw1 = _tc_linear1(degp, x, W1)
    s1 = _sc_segsum(hw1, psrc, pdst)
    hw2 = _tc_mid(degp, s1, b1, W2)
    s2 = _sc_segsum(hw2, psrc, pdst)
    h = _tc_final(degp, s2, b2)
    pos_score, neg_score = _sc_scores(h, psrc, pdst, nsrc, ndst)
    return (pos_score, neg_score)


# speedup vs baseline: 3.4936x; 3.4936x over previous
import dataclasses
import functools

import jax
import jax.numpy as jnp
from jax import lax
from jax.experimental import pallas as pl
from jax.experimental.pallas import tpu as pltpu
from jax.experimental.pallas import tpu_sc as plsc

N = 10000
NP = 10240
E = 320000
D = 128

NC = 2
NT = 16
EC = E // NC
ET = EC // NT
G = 80
NCH = ET // G
RT = NP // NT
ZR = 128

_MESH = plsc.VectorSubcoreMesh(
    core_axis_name="c", subcore_axis_name="s", num_cores=NC, num_subcores=NT
)

_SC_PARAMS = pltpu.CompilerParams()
if "needs_layout_passes" in pltpu.CompilerParams.__dataclass_fields__:
    _SC_PARAMS = dataclasses.replace(_SC_PARAMS, needs_layout_passes=False)

ROWBLK = 1024


@functools.partial(
    pl.kernel,
    out_type=jax.ShapeDtypeStruct((2, NC * NT, NP), jnp.float32),
    mesh=_MESH,
    compiler_params=_SC_PARAMS,
    scratch_types=[
        pltpu.VMEM((ET,), jnp.int32),
        pltpu.VMEM((NP,), jnp.float32),
        pltpu.VMEM((NP,), jnp.float32),
    ],
)
def _sc_degrees(src_hbm, dst_hbm, out_hbm, idx_v, hsrc_v, hdst_v):
    cid = lax.axis_index("c")
    sid = lax.axis_index("s")
    wid = cid * NT + sid
    base = wid * ET

    @pl.loop(0, NP, step=16)
    def _(i):
        hsrc_v[pl.ds(i, 16)] = jnp.zeros((16,), jnp.float32)
        hdst_v[pl.ds(i, 16)] = jnp.zeros((16,), jnp.float32)

    ones16 = jnp.ones((16,), jnp.float32)

    pltpu.sync_copy(src_hbm.at[pl.ds(base, ET)], idx_v)

    @pl.loop(0, ET, step=16)
    def _(i):
        plsc.addupdate_scatter(hsrc_v, [idx_v[pl.ds(i, 16)]], ones16)

    pltpu.sync_copy(dst_hbm.at[pl.ds(base, ET)], idx_v)

    @pl.loop(0, ET, step=16)
    def _(i):
        plsc.addupdate_scatter(hdst_v, [idx_v[pl.ds(i, 16)]], ones16)

    pltpu.sync_copy(hsrc_v, out_hbm.at[0, wid])
    pltpu.sync_copy(hdst_v, out_hbm.at[1, wid])


@functools.partial(
    pl.kernel,
    out_type=jax.ShapeDtypeStruct((NC, NP, D), jnp.float32),
    mesh=_MESH,
    compiler_params=_SC_PARAMS,
    scratch_types=[
        pltpu.VMEM((G,), jnp.int32),
        pltpu.VMEM((G,), jnp.int32),
        pltpu.VMEM((G, D), jnp.float32),
        pltpu.VMEM((ZR, D), jnp.float32),
        pltpu.VMEM_SHARED((NP, D), jnp.float32),
        pltpu.SemaphoreType.DMA,
    ],
)
def _sc_segsum(hw_hbm, src_hbm, dst_hbm, out_hbm,
               idx_s, idx_d, rows_v, zbuf, agg_sh, sem):
    cid = lax.axis_index("c")
    sid = lax.axis_index("s")

    @pl.loop(0, ZR)
    def _(i):
        @pl.loop(0, D, step=16)
        def _(j):
            zbuf[i, pl.ds(j, 16)] = jnp.zeros((16,), jnp.float32)

    @pl.loop(0, RT // ZR)
    def _(j):
        pltpu.sync_copy(zbuf, agg_sh.at[pl.ds(sid * RT + j * ZR, ZR)])

    plsc.subcore_barrier()

    base = cid * EC + sid * ET

    @pl.loop(0, NCH)
    def _(ch):
        pltpu.sync_copy(src_hbm.at[pl.ds(base + ch * G, G)], idx_s)
        pltpu.sync_copy(dst_hbm.at[pl.ds(base + ch * G, G)], idx_d)
        pltpu.async_copy(hw_hbm.at[idx_s], rows_v, sem).wait()
        pltpu.sync_copy(rows_v, agg_sh.at[idx_d], add=True)

    plsc.subcore_barrier()

    pltpu.sync_copy(agg_sh.at[pl.ds(sid * RT, RT)],
                    out_hbm.at[cid, pl.ds(sid * RT, RT)])


@functools.partial(
    pl.kernel,
    out_type=[
        jax.ShapeDtypeStruct((E,), jnp.float32),
        jax.ShapeDtypeStruct((E,), jnp.float32),
    ],
    mesh=_MESH,
    compiler_params=_SC_PARAMS,
    scratch_types=[
        pltpu.VMEM((G,), jnp.int32),
        pltpu.VMEM((G,), jnp.int32),
        pltpu.VMEM((G, D), jnp.float32),
        pltpu.VMEM((G, D), jnp.float32),
        pltpu.VMEM((ET,), jnp.float32),
        pltpu.SemaphoreType.DMA,
        pltpu.SemaphoreType.DMA,
    ],
)
def _sc_scores(h_hbm, psrc, pdst, nsrc, ndst, pos_out, neg_out,
               idx_s, idx_d, hs, hd, score_v, sem1, sem2):
    cid = lax.axis_index("c")
    sid = lax.axis_index("s")
    base = cid * EC + sid * ET

    def one_set(src_arr, dst_arr, out_arr):
        @pl.loop(0, NCH)
        def _(ch):
            pltpu.sync_copy(src_arr.at[pl.ds(base + ch * G, G)], idx_s)
            pltpu.sync_copy(dst_arr.at[pl.ds(base + ch * G, G)], idx_d)
            cp1 = pltpu.async_copy(h_hbm.at[idx_s], hs, sem1)
            cp2 = pltpu.async_copy(h_hbm.at[idx_d], hd, sem2)
            cp1.wait()
            cp2.wait()

            lanes = jnp.arange(16, dtype=jnp.int32)

            @pl.loop(0, G, step=16)
            def _(e0):
                vec = jnp.zeros((16,), jnp.float32)
                for l in range(16):
                    e = e0 + l
                    acc = hs[e, pl.ds(0, 16)] * hd[e, pl.ds(0, 16)]
                    for j in range(1, D // 16):
                        acc = acc + hs[e, pl.ds(j * 16, 16)] * hd[e, pl.ds(j * 16, 16)]
                    vec = jnp.where(lanes == l, jnp.sum(acc), vec)
                score_v[pl.ds(ch * G + e0, 16)] = vec

        pltpu.sync_copy(score_v, out_arr.at[pl.ds(base, ET)])

    one_set(psrc, pdst, pos_out)
    one_set(nsrc, ndst, neg_out)


def _tc1_body(deg_ref, x_ref, w_ref, out_ref):
    dout = jnp.sum(deg_ref[0], axis=0)
    ns = lax.rsqrt(jnp.maximum(dout, 1.0))[:, None]
    xw = jnp.dot(x_ref[...], w_ref[...], preferred_element_type=jnp.float32)
    out_ref[...] = xw * ns


def _tc_linear1(degp, x, W1):
    return pl.pallas_call(
        _tc1_body,
        grid=(NP // ROWBLK,),
        in_specs=[
            pl.BlockSpec((2, NC * NT, ROWBLK), lambda i: (0, 0, i)),
            pl.BlockSpec((ROWBLK, D), lambda i: (i, 0)),
            pl.BlockSpec((D, D), lambda i: (0, 0)),
        ],
        out_specs=pl.BlockSpec((ROWBLK, D), lambda i: (i, 0)),
        out_shape=jax.ShapeDtypeStruct((NP, D), jnp.float32),
    )(degp, x, W1)


def _tc2_body(deg_ref, s_ref, b_ref, w_ref, out_ref):
    nd = lax.rsqrt(jnp.maximum(jnp.sum(deg_ref[1], axis=0), 1.0))[:, None]
    ns = lax.rsqrt(jnp.maximum(jnp.sum(deg_ref[0], axis=0), 1.0))[:, None]
    h1 = jnp.maximum((s_ref[0] + s_ref[1]) * nd + b_ref[...], 0.0)
    hw = jnp.dot(h1, w_ref[...], preferred_element_type=jnp.float32)
    out_ref[...] = hw * ns


def _tc_mid(degp, s1, b1, W2):
    return pl.pallas_call(
        _tc2_body,
        grid=(NP // ROWBLK,),
        in_specs=[
            pl.BlockSpec((2, NC * NT, ROWBLK), lambda i: (0, 0, i)),
            pl.BlockSpec((NC, ROWBLK, D), lambda i: (0, i, 0)),
            pl.BlockSpec((1, D), lambda i: (0, 0)),
            pl.BlockSpec((D, D), lambda i: (0, 0)),
        ],
        out_specs=pl.BlockSpec((ROWBLK, D), lambda i: (i, 0)),
        out_shape=jax.ShapeDtypeStruct((NP, D), jnp.float32),
    )(degp, s1, b1.reshape(1, D), W2)


def _tc3_body(deg_ref, s_ref, b_ref, out_ref):
    nd = lax.rsqrt(jnp.maximum(jnp.sum(deg_ref[1], axis=0), 1.0))[:, None]
    out_ref[...] = (s_ref[0] + s_ref[1]) * nd + b_ref[...]


def _tc_final(degp, s2, b2):
    return pl.pallas_call(
        _tc3_body,
        grid=(NP // ROWBLK,),
        in_specs=[
            pl.BlockSpec((2, NC * NT, ROWBLK), lambda i: (0, 0, i)),
            pl.BlockSpec((NC, ROWBLK, D), lambda i: (0, i, 0)),
            pl.BlockSpec((1, D), lambda i: (0, 0)),
        ],
        out_specs=pl.BlockSpec((ROWBLK, D), lambda i: (i, 0)),
        out_shape=jax.ShapeDtypeStruct((NP, D), jnp.float32),
    )(degp, s2, b2.reshape(1, D))


def kernel(x, pos_edge_index, neg_edge_index, W1, b1, W2, b2):
    psrc = pos_edge_index[0].astype(jnp.int32)
    pdst = pos_edge_index[1].astype(jnp.int32)
    nsrc = neg_edge_index[0].astype(jnp.int32)
    ndst = neg_edge_index[1].astype(jnp.int32)

    xp = jnp.pad(x, ((0, NP - N), (0, 0)))

    degp = _sc_degrees(psrc, pdst)
    hw1 = _tc_linear1(degp, xp, W1)
    s1 = _sc_segsum(hw1, psrc, pdst)
    hw2 = _tc_mid(degp, s1, b1, W2)
    s2 = _sc_segsum(hw2, psrc, pdst)
    h = _tc_final(degp, s2, b2)
    pos_score, neg_score = _sc_scores(h, psrc, pdst, nsrc, ndst)
    return (pos_score, neg_score)


# speedup vs baseline: 6.0900x; 1.7432x over previous
import dataclasses
import functools

import jax
import jax.numpy as jnp
from jax import lax
from jax.experimental import pallas as pl
from jax.experimental.pallas import tpu as pltpu
from jax.experimental.pallas import tpu_sc as plsc

N = 10000
NP = 10240
E = 320000
D = 128

NC = 2
NT = 16
EC = E // NC
ET = EC // NT
G = 80
NCH = ET // G
RT = NP // NT
ZR = 128

_MESH = plsc.VectorSubcoreMesh(
    core_axis_name="c", subcore_axis_name="s", num_cores=NC, num_subcores=NT
)

_SC_PARAMS = pltpu.CompilerParams()
if "needs_layout_passes" in pltpu.CompilerParams.__dataclass_fields__:
    _SC_PARAMS = dataclasses.replace(_SC_PARAMS, needs_layout_passes=False)

ROWBLK = 1024


@functools.partial(
    pl.kernel,
    out_type=jax.ShapeDtypeStruct((2, NC * NT, NP), jnp.float32),
    mesh=_MESH,
    compiler_params=_SC_PARAMS,
    scratch_types=[
        pltpu.VMEM((ET,), jnp.int32),
        pltpu.VMEM((NP,), jnp.float32),
        pltpu.VMEM((NP,), jnp.float32),
    ],
)
def _sc_degrees(src_hbm, dst_hbm, out_hbm, idx_v, hsrc_v, hdst_v):
    cid = lax.axis_index("c")
    sid = lax.axis_index("s")
    wid = cid * NT + sid
    base = wid * ET

    @pl.loop(0, NP, step=16)
    def _(i):
        hsrc_v[pl.ds(i, 16)] = jnp.zeros((16,), jnp.float32)
        hdst_v[pl.ds(i, 16)] = jnp.zeros((16,), jnp.float32)

    ones16 = jnp.ones((16,), jnp.float32)

    pltpu.sync_copy(src_hbm.at[pl.ds(base, ET)], idx_v)

    @pl.loop(0, ET, step=16)
    def _(i):
        plsc.addupdate_scatter(hsrc_v, [idx_v[pl.ds(i, 16)]], ones16)

    pltpu.sync_copy(dst_hbm.at[pl.ds(base, ET)], idx_v)

    @pl.loop(0, ET, step=16)
    def _(i):
        plsc.addupdate_scatter(hdst_v, [idx_v[pl.ds(i, 16)]], ones16)

    pltpu.sync_copy(hsrc_v, out_hbm.at[0, wid])
    pltpu.sync_copy(hdst_v, out_hbm.at[1, wid])


@functools.partial(
    pl.kernel,
    out_type=jax.ShapeDtypeStruct((NC, NP, D), jnp.float32),
    mesh=_MESH,
    compiler_params=_SC_PARAMS,
    scratch_types=[
        pltpu.VMEM((ET,), jnp.int32),
        pltpu.VMEM((ET,), jnp.int32),
        pltpu.VMEM((G, D), jnp.float32),
        pltpu.VMEM((G, D), jnp.float32),
        pltpu.VMEM_SHARED((NP, D), jnp.float32),
        pltpu.SemaphoreType.DMA,
        pltpu.SemaphoreType.DMA,
    ],
)
def _sc_segsum(hw_hbm, src_hbm, dst_hbm, out_hbm,
               idx_s, idx_d, rows0, rows1, agg_sh, sem0, sem1):
    cid = lax.axis_index("c")
    sid = lax.axis_index("s")

    @pl.loop(0, G)
    def _(i):
        @pl.loop(0, D, step=16)
        def _(j):
            rows0[i, pl.ds(j, 16)] = jnp.zeros((16,), jnp.float32)

    @pl.loop(0, RT // G)
    def _(j):
        pltpu.sync_copy(rows0, agg_sh.at[pl.ds(sid * RT + j * G, G)])

    base = cid * EC + sid * ET
    pltpu.sync_copy(src_hbm.at[pl.ds(base, ET)], idx_s)
    pltpu.sync_copy(dst_hbm.at[pl.ds(base, ET)], idx_d)

    plsc.subcore_barrier()

    def gat(ch, buf, sem):
        pltpu.async_copy(hw_hbm.at[idx_s.at[pl.ds(ch * G, G)]], buf, sem)

    def wat(ch, buf, sem):
        pltpu.make_async_copy(
            hw_hbm.at[idx_s.at[pl.ds(ch * G, G)]], buf, sem).wait()

    def acc(ch, buf):
        pltpu.sync_copy(buf, agg_sh.at[idx_d.at[pl.ds(ch * G, G)]], add=True)

    gat(0, rows0, sem0)
    gat(1, rows1, sem1)

    @pl.loop(0, NCH - 1, step=2)
    def _(c):
        wat(c, rows0, sem0)
        acc(c, rows0)
        gat(c + 2, rows0, sem0)
        wat(c + 1, rows1, sem1)
        acc(c + 1, rows1)

        @pl.when(c < NCH - 3)
        def _():
            gat(c + 3, rows1, sem1)

    wat(NCH - 1, rows0, sem0)
    acc(NCH - 1, rows0)

    plsc.subcore_barrier()

    pltpu.sync_copy(agg_sh.at[pl.ds(sid * RT, RT)],
                    out_hbm.at[cid, pl.ds(sid * RT, RT)])


@functools.partial(
    pl.kernel,
    out_type=[
        jax.ShapeDtypeStruct((E,), jnp.float32),
        jax.ShapeDtypeStruct((E,), jnp.float32),
    ],
    mesh=_MESH,
    compiler_params=_SC_PARAMS,
    scratch_types=[
        pltpu.VMEM((ET,), jnp.int32),
        pltpu.VMEM((ET,), jnp.int32),
        pltpu.VMEM((G, D), jnp.float32),
        pltpu.VMEM((G, D), jnp.float32),
        pltpu.VMEM((G, D), jnp.float32),
        pltpu.VMEM((G, D), jnp.float32),
        pltpu.VMEM((ET,), jnp.float32),
        pltpu.SemaphoreType.DMA,
        pltpu.SemaphoreType.DMA,
    ],
)
def _sc_scores(h_hbm, psrc, pdst, nsrc, ndst, pos_out, neg_out,
               idx_s, idx_d, hs0, hd0, hs1, hd1, score_v, sem0, sem1):
    cid = lax.axis_index("c")
    sid = lax.axis_index("s")
    base = cid * EC + sid * ET
    lanes = jnp.arange(16, dtype=jnp.int32)

    def one_set(src_arr, dst_arr, out_arr):
        pltpu.sync_copy(src_arr.at[pl.ds(base, ET)], idx_s)
        pltpu.sync_copy(dst_arr.at[pl.ds(base, ET)], idx_d)

        def gat(ch, hs, hd, sem):
            pltpu.async_copy(h_hbm.at[idx_s.at[pl.ds(ch * G, G)]], hs, sem)
            pltpu.async_copy(h_hbm.at[idx_d.at[pl.ds(ch * G, G)]], hd, sem)

        def wat(ch, hs, hd, sem):
            pltpu.make_async_copy(
                h_hbm.at[idx_s.at[pl.ds(ch * G, G)]], hs, sem).wait()
            pltpu.make_async_copy(
                h_hbm.at[idx_d.at[pl.ds(ch * G, G)]], hd, sem).wait()

        def dots(ch, hs, hd):
            @pl.loop(0, G, step=16)
            def _(e0):
                vec = jnp.zeros((16,), jnp.float32)
                for l in range(16):
                    e = e0 + l
                    acc = hs[e, pl.ds(0, 16)] * hd[e, pl.ds(0, 16)]
                    for j in range(1, D // 16):
                        acc = acc + hs[e, pl.ds(j * 16, 16)] * hd[e, pl.ds(j * 16, 16)]
                    vec = jnp.where(lanes == l, jnp.sum(acc), vec)
                score_v[pl.ds(ch * G + e0, 16)] = vec

        gat(0, hs0, hd0, sem0)
        gat(1, hs1, hd1, sem1)

        @pl.loop(0, NCH - 1, step=2)
        def _(c):
            wat(c, hs0, hd0, sem0)
            dots(c, hs0, hd0)
            gat(c + 2, hs0, hd0, sem0)
            wat(c + 1, hs1, hd1, sem1)
            dots(c + 1, hs1, hd1)

            @pl.when(c < NCH - 3)
            def _():
                gat(c + 3, hs1, hd1, sem1)

        wat(NCH - 1, hs0, hd0, sem0)
        dots(NCH - 1, hs0, hd0)

        pltpu.sync_copy(score_v, out_arr.at[pl.ds(base, ET)])

    one_set(psrc, pdst, pos_out)
    one_set(nsrc, ndst, neg_out)


def _tc1_body(deg_ref, x_ref, w_ref, out_ref):
    dout = jnp.sum(deg_ref[0], axis=0)
    ns = lax.rsqrt(jnp.maximum(dout, 1.0))[:, None]
    xw = jnp.dot(x_ref[...], w_ref[...], preferred_element_type=jnp.float32)
    out_ref[...] = xw * ns


def _tc_linear1(degp, x, W1):
    return pl.pallas_call(
        _tc1_body,
        grid=(NP // ROWBLK,),
        in_specs=[
            pl.BlockSpec((2, NC * NT, ROWBLK), lambda i: (0, 0, i)),
            pl.BlockSpec((ROWBLK, D), lambda i: (i, 0)),
            pl.BlockSpec((D, D), lambda i: (0, 0)),
        ],
        out_specs=pl.BlockSpec((ROWBLK, D), lambda i: (i, 0)),
        out_shape=jax.ShapeDtypeStruct((NP, D), jnp.float32),
    )(degp, x, W1)


def _tc2_body(deg_ref, s_ref, b_ref, w_ref, out_ref):
    nd = lax.rsqrt(jnp.maximum(jnp.sum(deg_ref[1], axis=0), 1.0))[:, None]
    ns = lax.rsqrt(jnp.maximum(jnp.sum(deg_ref[0], axis=0), 1.0))[:, None]
    h1 = jnp.maximum((s_ref[0] + s_ref[1]) * nd + b_ref[...], 0.0)
    hw = jnp.dot(h1, w_ref[...], preferred_element_type=jnp.float32)
    out_ref[...] = hw * ns


def _tc_mid(degp, s1, b1, W2):
    return pl.pallas_call(
        _tc2_body,
        grid=(NP // ROWBLK,),
        in_specs=[
            pl.BlockSpec((2, NC * NT, ROWBLK), lambda i: (0, 0, i)),
            pl.BlockSpec((NC, ROWBLK, D), lambda i: (0, i, 0)),
            pl.BlockSpec((1, D), lambda i: (0, 0)),
            pl.BlockSpec((D, D), lambda i: (0, 0)),
        ],
        out_specs=pl.BlockSpec((ROWBLK, D), lambda i: (i, 0)),
        out_shape=jax.ShapeDtypeStruct((NP, D), jnp.float32),
    )(degp, s1, b1.reshape(1, D), W2)


def _tc3_body(deg_ref, s_ref, b_ref, out_ref):
    nd = lax.rsqrt(jnp.maximum(jnp.sum(deg_ref[1], axis=0), 1.0))[:, None]
    out_ref[...] = (s_ref[0] + s_ref[1]) * nd + b_ref[...]


def _tc_final(degp, s2, b2):
    return pl.pallas_call(
        _tc3_body,
        grid=(NP // ROWBLK,),
        in_specs=[
            pl.BlockSpec((2, NC * NT, ROWBLK), lambda i: (0, 0, i)),
            pl.BlockSpec((NC, ROWBLK, D), lambda i: (0, i, 0)),
            pl.BlockSpec((1, D), lambda i: (0, 0)),
        ],
        out_specs=pl.BlockSpec((ROWBLK, D), lambda i: (i, 0)),
        out_shape=jax.ShapeDtypeStruct((NP, D), jnp.float32),
    )(degp, s2, b2.reshape(1, D))


def kernel(x, pos_edge_index, neg_edge_index, W1, b1, W2, b2):
    psrc = pos_edge_index[0].astype(jnp.int32)
    pdst = pos_edge_index[1].astype(jnp.int32)
    nsrc = neg_edge_index[0].astype(jnp.int32)
    ndst = neg_edge_index[1].astype(jnp.int32)

    xp = jnp.pad(x, ((0, NP - N), (0, 0)))

    degp = _sc_degrees(psrc, pdst)
    hw1 = _tc_linear1(degp, xp, W1)
    s1 = _sc_segsum(hw1, psrc, pdst)
    hw2 = _tc_mid(degp, s1, b1, W2)
    s2 = _sc_segsum(hw2, psrc, pdst)
    h = _tc_final(degp, s2, b2)
    pos_score, neg_score = _sc_scores(h, psrc, pdst, nsrc, ndst)
    return (pos_score, neg_score)


# speedup vs baseline: 6.6582x; 1.0933x over previous
import dataclasses
import functools

import jax
import jax.numpy as jnp
from jax import lax
from jax.experimental import pallas as pl
from jax.experimental.pallas import tpu as pltpu
from jax.experimental.pallas import tpu_sc as plsc

N = 10000
NP = 10240
E = 320000
D = 128

NC = 2
NT = 16
EC = E // NC
ET = EC // NT
G = 80
NCH = ET // G
RT = NP // NT
ZR = 128

_MESH = plsc.VectorSubcoreMesh(
    core_axis_name="c", subcore_axis_name="s", num_cores=NC, num_subcores=NT
)

_SC_PARAMS = pltpu.CompilerParams()
if "needs_layout_passes" in pltpu.CompilerParams.__dataclass_fields__:
    _SC_PARAMS = dataclasses.replace(_SC_PARAMS, needs_layout_passes=False)

ROWBLK = 1024


@functools.partial(
    pl.kernel,
    out_type=jax.ShapeDtypeStruct((2, NC * NT, NP), jnp.float32),
    mesh=_MESH,
    compiler_params=_SC_PARAMS,
    scratch_types=[
        pltpu.VMEM((ET,), jnp.int32),
        pltpu.VMEM((NP,), jnp.float32),
        pltpu.VMEM((NP,), jnp.float32),
    ],
)
def _sc_degrees(src_hbm, dst_hbm, out_hbm, idx_v, hsrc_v, hdst_v):
    cid = lax.axis_index("c")
    sid = lax.axis_index("s")
    wid = cid * NT + sid
    base = wid * ET

    @pl.loop(0, NP, step=16)
    def _(i):
        hsrc_v[pl.ds(i, 16)] = jnp.zeros((16,), jnp.float32)
        hdst_v[pl.ds(i, 16)] = jnp.zeros((16,), jnp.float32)

    ones16 = jnp.ones((16,), jnp.float32)

    pltpu.sync_copy(src_hbm.at[pl.ds(base, ET)], idx_v)

    @pl.loop(0, ET, step=16)
    def _(i):
        plsc.addupdate_scatter(hsrc_v, [idx_v[pl.ds(i, 16)]], ones16)

    pltpu.sync_copy(dst_hbm.at[pl.ds(base, ET)], idx_v)

    @pl.loop(0, ET, step=16)
    def _(i):
        plsc.addupdate_scatter(hdst_v, [idx_v[pl.ds(i, 16)]], ones16)

    pltpu.sync_copy(hsrc_v, out_hbm.at[0, wid])
    pltpu.sync_copy(hdst_v, out_hbm.at[1, wid])


@functools.partial(
    pl.kernel,
    out_type=jax.ShapeDtypeStruct((NC, NP, D), jnp.float32),
    mesh=_MESH,
    compiler_params=_SC_PARAMS,
    scratch_types=[
        pltpu.VMEM((ET,), jnp.int32),
        pltpu.VMEM((ET,), jnp.int32),
        pltpu.VMEM((G, D), jnp.float32),
        pltpu.VMEM((G, D), jnp.float32),
        pltpu.VMEM_SHARED((NP, D), jnp.float32),
        pltpu.SemaphoreType.DMA,
        pltpu.SemaphoreType.DMA,
        pltpu.SemaphoreType.DMA,
        pltpu.SemaphoreType.DMA,
    ],
)
def _sc_segsum(hw_hbm, src_hbm, dst_hbm, out_hbm,
               idx_s, idx_d, rows0, rows1, agg_sh, sem0, sem1, ssem0, ssem1):
    cid = lax.axis_index("c")
    sid = lax.axis_index("s")

    @pl.loop(0, G)
    def _(i):
        @pl.loop(0, D, step=16)
        def _(j):
            rows0[i, pl.ds(j, 16)] = jnp.zeros((16,), jnp.float32)

    @pl.loop(0, RT // G)
    def _(j):
        pltpu.sync_copy(rows0, agg_sh.at[pl.ds(sid * RT + j * G, G)])

    base = cid * EC + sid * ET
    pltpu.sync_copy(src_hbm.at[pl.ds(base, ET)], idx_s)
    pltpu.sync_copy(dst_hbm.at[pl.ds(base, ET)], idx_d)

    plsc.subcore_barrier()

    def gat(ch, buf, sem):
        pltpu.async_copy(hw_hbm.at[idx_s.at[pl.ds(ch * G, G)]], buf, sem)

    def wat(ch, buf, sem):
        pltpu.make_async_copy(
            hw_hbm.at[idx_s.at[pl.ds(ch * G, G)]], buf, sem).wait()

    def acc(ch, buf, sem):
        pltpu.async_copy(buf, agg_sh.at[idx_d.at[pl.ds(ch * G, G)]], sem,
                         add=True)

    def acc_wait(ch, buf, sem):
        pltpu.make_async_copy(
            buf, agg_sh.at[idx_d.at[pl.ds(ch * G, G)]], sem).wait()

    gat(0, rows0, sem0)
    gat(1, rows1, sem1)

    @pl.loop(0, NCH - 1, step=2)
    def _(c):
        wat(c, rows0, sem0)
        acc(c, rows0, ssem0)
        wat(c + 1, rows1, sem1)
        acc(c + 1, rows1, ssem1)
        acc_wait(c, rows0, ssem0)
        gat(c + 2, rows0, sem0)
        acc_wait(c + 1, rows1, ssem1)

        @pl.when(c < NCH - 3)
        def _():
            gat(c + 3, rows1, sem1)

    wat(NCH - 1, rows0, sem0)
    acc(NCH - 1, rows0, ssem0)
    acc_wait(NCH - 1, rows0, ssem0)

    plsc.subcore_barrier()

    pltpu.sync_copy(agg_sh.at[pl.ds(sid * RT, RT)],
                    out_hbm.at[cid, pl.ds(sid * RT, RT)])


@functools.partial(
    pl.kernel,
    out_type=[
        jax.ShapeDtypeStruct((E,), jnp.float32),
        jax.ShapeDtypeStruct((E,), jnp.float32),
    ],
    mesh=_MESH,
    compiler_params=_SC_PARAMS,
    scratch_types=[
        pltpu.VMEM((ET,), jnp.int32),
        pltpu.VMEM((ET,), jnp.int32),
        pltpu.VMEM((G, D), jnp.float32),
        pltpu.VMEM((G, D), jnp.float32),
        pltpu.VMEM((G, D), jnp.float32),
        pltpu.VMEM((G, D), jnp.float32),
        pltpu.VMEM((ET,), jnp.float32),
        pltpu.SemaphoreType.DMA,
        pltpu.SemaphoreType.DMA,
    ],
)
def _sc_scores(h_hbm, psrc, pdst, nsrc, ndst, pos_out, neg_out,
               idx_s, idx_d, hs0, hd0, hs1, hd1, score_v, sem0, sem1):
    cid = lax.axis_index("c")
    sid = lax.axis_index("s")
    base = cid * EC + sid * ET
    lanes = jnp.arange(16, dtype=jnp.int32)
    _dn = lax.GatherDimensionNumbers(
        offset_dims=(), collapsed_slice_dims=(0,), start_index_map=(0,))

    def shuffle(x, sh):
        return lax.gather(x, ((lanes ^ sh)[:, None]), _dn, slice_sizes=(1,),
                          mode=lax.GatherScatterMode.PROMISE_IN_BOUNDS)

    def one_set(src_arr, dst_arr, out_arr):
        pltpu.sync_copy(src_arr.at[pl.ds(base, ET)], idx_s)
        pltpu.sync_copy(dst_arr.at[pl.ds(base, ET)], idx_d)

        def gat(ch, hs, hd, sem):
            pltpu.async_copy(h_hbm.at[idx_s.at[pl.ds(ch * G, G)]], hs, sem)
            pltpu.async_copy(h_hbm.at[idx_d.at[pl.ds(ch * G, G)]], hd, sem)

        def wat(ch, hs, hd, sem):
            pltpu.make_async_copy(
                h_hbm.at[idx_s.at[pl.ds(ch * G, G)]], hs, sem).wait()
            pltpu.make_async_copy(
                h_hbm.at[idx_d.at[pl.ds(ch * G, G)]], hd, sem).wait()

        def dots(ch, hs, hd):
            @pl.loop(0, G, step=16)
            def _(e0):
                vec = jnp.zeros((16,), jnp.float32)
                for l in range(16):
                    e = e0 + l
                    acc = hs[e, pl.ds(0, 16)] * hd[e, pl.ds(0, 16)]
                    for j in range(1, D // 16):
                        acc = acc + hs[e, pl.ds(j * 16, 16)] * hd[e, pl.ds(j * 16, 16)]
                    for sh in (8, 4, 2, 1):
                        acc = acc + shuffle(acc, sh)
                    vec = jnp.where(lanes == l, acc, vec)
                score_v[pl.ds(ch * G + e0, 16)] = vec

        gat(0, hs0, hd0, sem0)
        gat(1, hs1, hd1, sem1)

        @pl.loop(0, NCH - 1, step=2)
        def _(c):
            wat(c, hs0, hd0, sem0)
            dots(c, hs0, hd0)
            gat(c + 2, hs0, hd0, sem0)
            wat(c + 1, hs1, hd1, sem1)
            dots(c + 1, hs1, hd1)

            @pl.when(c < NCH - 3)
            def _():
                gat(c + 3, hs1, hd1, sem1)

        wat(NCH - 1, hs0, hd0, sem0)
        dots(NCH - 1, hs0, hd0)

        pltpu.sync_copy(score_v, out_arr.at[pl.ds(base, ET)])

    one_set(psrc, pdst, pos_out)
    one_set(nsrc, ndst, neg_out)


def _tc1_body(deg_ref, x_ref, w_ref, out_ref):
    dout = jnp.sum(deg_ref[0], axis=0)
    ns = lax.rsqrt(jnp.maximum(dout, 1.0))[:, None]
    xw = jnp.dot(x_ref[...], w_ref[...], preferred_element_type=jnp.float32)
    out_ref[...] = xw * ns


def _tc_linear1(degp, x, W1):
    return pl.pallas_call(
        _tc1_body,
        grid=(NP // ROWBLK,),
        in_specs=[
            pl.BlockSpec((2, NC * NT, ROWBLK), lambda i: (0, 0, i)),
            pl.BlockSpec((ROWBLK, D), lambda i: (i, 0)),
            pl.BlockSpec((D, D), lambda i: (0, 0)),
        ],
        out_specs=pl.BlockSpec((ROWBLK, D), lambda i: (i, 0)),
        out_shape=jax.ShapeDtypeStruct((NP, D), jnp.float32),
    )(degp, x, W1)


def _tc2_body(deg_ref, s_ref, b_ref, w_ref, out_ref):
    nd = lax.rsqrt(jnp.maximum(jnp.sum(deg_ref[1], axis=0), 1.0))[:, None]
    ns = lax.rsqrt(jnp.maximum(jnp.sum(deg_ref[0], axis=0), 1.0))[:, None]
    h1 = jnp.maximum((s_ref[0] + s_ref[1]) * nd + b_ref[...], 0.0)
    hw = jnp.dot(h1, w_ref[...], preferred_element_type=jnp.float32)
    out_ref[...] = hw * ns


def _tc_mid(degp, s1, b1, W2):
    return pl.pallas_call(
        _tc2_body,
        grid=(NP // ROWBLK,),
        in_specs=[
            pl.BlockSpec((2, NC * NT, ROWBLK), lambda i: (0, 0, i)),
            pl.BlockSpec((NC, ROWBLK, D), lambda i: (0, i, 0)),
            pl.BlockSpec((1, D), lambda i: (0, 0)),
            pl.BlockSpec((D, D), lambda i: (0, 0)),
        ],
        out_specs=pl.BlockSpec((ROWBLK, D), lambda i: (i, 0)),
        out_shape=jax.ShapeDtypeStruct((NP, D), jnp.float32),
    )(degp, s1, b1.reshape(1, D), W2)


def _tc3_body(deg_ref, s_ref, b_ref, out_ref):
    nd = lax.rsqrt(jnp.maximum(jnp.sum(deg_ref[1], axis=0), 1.0))[:, None]
    out_ref[...] = (s_ref[0] + s_ref[1]) * nd + b_ref[...]


def _tc_final(degp, s2, b2):
    return pl.pallas_call(
        _tc3_body,
        grid=(NP // ROWBLK,),
        in_specs=[
            pl.BlockSpec((2, NC * NT, ROWBLK), lambda i: (0, 0, i)),
            pl.BlockSpec((NC, ROWBLK, D), lambda i: (0, i, 0)),
            pl.BlockSpec((1, D), lambda i: (0, 0)),
        ],
        out_specs=pl.BlockSpec((ROWBLK, D), lambda i: (i, 0)),
        out_shape=jax.ShapeDtypeStruct((NP, D), jnp.float32),
    )(degp, s2, b2.reshape(1, D))


def kernel(x, pos_edge_index, neg_edge_index, W1, b1, W2, b2):
    psrc = pos_edge_index[0].astype(jnp.int32)
    pdst = pos_edge_index[1].astype(jnp.int32)
    nsrc = neg_edge_index[0].astype(jnp.int32)
    ndst = neg_edge_index[1].astype(jnp.int32)

    xp = jnp.pad(x, ((0, NP - N), (0, 0)))

    degp = _sc_degrees(psrc, pdst)
    hw1 = _tc_linear1(degp, xp, W1)
    s1 = _sc_segsum(hw1, psrc, pdst)
    hw2 = _tc_mid(degp, s1, b1, W2)
    s2 = _sc_segsum(hw2, psrc, pdst)
    h = _tc_final(degp, s2, b2)
    pos_score, neg_score = _sc_scores(h, psrc, pdst, nsrc, ndst)
    return (pos_score, neg_score)


# speedup vs baseline: 7.1002x; 1.0664x over previous
import dataclasses
import functools

import jax
import jax.numpy as jnp
from jax import lax
from jax.experimental import pallas as pl
from jax.experimental.pallas import tpu as pltpu
from jax.experimental.pallas import tpu_sc as plsc

N = 10000
NP = 10240
E = 320000
D = 128

NC = 2
NT = 16
EC = E // NC
ET = EC // NT
G = 80
NCH = ET // G
GS = 40
NCHS = ET // GS
RT = NP // NT
ZR = 128

_MESH = plsc.VectorSubcoreMesh(
    core_axis_name="c", subcore_axis_name="s", num_cores=NC, num_subcores=NT
)

_SC_PARAMS = pltpu.CompilerParams()
if "needs_layout_passes" in pltpu.CompilerParams.__dataclass_fields__:
    _SC_PARAMS = dataclasses.replace(_SC_PARAMS, needs_layout_passes=False)

ROWBLK = 1024


@functools.partial(
    pl.kernel,
    out_type=jax.ShapeDtypeStruct((2, NC * NT, NP), jnp.float32),
    mesh=_MESH,
    compiler_params=_SC_PARAMS,
    scratch_types=[
        pltpu.VMEM((ET,), jnp.int32),
        pltpu.VMEM((NP,), jnp.float32),
        pltpu.VMEM((NP,), jnp.float32),
    ],
)
def _sc_degrees(src_hbm, dst_hbm, out_hbm, idx_v, hsrc_v, hdst_v):
    cid = lax.axis_index("c")
    sid = lax.axis_index("s")
    wid = cid * NT + sid
    base = wid * ET

    @pl.loop(0, NP, step=16)
    def _(i):
        hsrc_v[pl.ds(i, 16)] = jnp.zeros((16,), jnp.float32)
        hdst_v[pl.ds(i, 16)] = jnp.zeros((16,), jnp.float32)

    ones16 = jnp.ones((16,), jnp.float32)

    pltpu.sync_copy(src_hbm.at[pl.ds(base, ET)], idx_v)

    @pl.loop(0, ET, step=16)
    def _(i):
        plsc.addupdate_scatter(hsrc_v, [idx_v[pl.ds(i, 16)]], ones16)

    pltpu.sync_copy(dst_hbm.at[pl.ds(base, ET)], idx_v)

    @pl.loop(0, ET, step=16)
    def _(i):
        plsc.addupdate_scatter(hdst_v, [idx_v[pl.ds(i, 16)]], ones16)

    pltpu.sync_copy(hsrc_v, out_hbm.at[0, wid])
    pltpu.sync_copy(hdst_v, out_hbm.at[1, wid])


@functools.partial(
    pl.kernel,
    out_type=jax.ShapeDtypeStruct((NC, NP, D), jnp.float32),
    mesh=_MESH,
    compiler_params=_SC_PARAMS,
    scratch_types=[
        pltpu.VMEM((ET,), jnp.int32),
        pltpu.VMEM((ET,), jnp.int32),
        pltpu.VMEM((GS, D), jnp.float32),
        pltpu.VMEM((GS, D), jnp.float32),
        pltpu.VMEM((GS, D), jnp.float32),
        pltpu.VMEM((GS, D), jnp.float32),
        pltpu.SemaphoreType.DMA,
        pltpu.SemaphoreType.DMA,
        pltpu.SemaphoreType.DMA,
        pltpu.SemaphoreType.DMA,
        pltpu.SemaphoreType.DMA,
        pltpu.SemaphoreType.DMA,
        pltpu.SemaphoreType.DMA,
        pltpu.SemaphoreType.DMA,
        pltpu.VMEM_SHARED((NP, D), jnp.float32),
    ],
)
def _sc_segsum(hw_hbm, src_hbm, dst_hbm, out_hbm,
               idx_s, idx_d, r0, r1, r2, r3,
               g0, g1, g2, g3, s0, s1, s2, s3, agg_sh):
    cid = lax.axis_index("c")
    sid = lax.axis_index("s")
    bufs = (r0, r1, r2, r3)
    gsems = (g0, g1, g2, g3)
    ssems = (s0, s1, s2, s3)

    @pl.loop(0, GS)
    def _(i):
        @pl.loop(0, D, step=16)
        def _(j):
            r0[i, pl.ds(j, 16)] = jnp.zeros((16,), jnp.float32)

    @pl.loop(0, RT // GS)
    def _(j):
        pltpu.sync_copy(r0, agg_sh.at[pl.ds(sid * RT + j * GS, GS)])

    base = cid * EC + sid * ET
    pltpu.sync_copy(src_hbm.at[pl.ds(base, ET)], idx_s)
    pltpu.sync_copy(dst_hbm.at[pl.ds(base, ET)], idx_d)

    plsc.subcore_barrier()

    def gat(ch, k):
        pltpu.async_copy(hw_hbm.at[idx_s.at[pl.ds(ch * GS, GS)]],
                         bufs[k], gsems[k])

    def wat(ch, k):
        pltpu.make_async_copy(hw_hbm.at[idx_s.at[pl.ds(ch * GS, GS)]],
                              bufs[k], gsems[k]).wait()

    def acc(ch, k):
        pltpu.async_copy(bufs[k], agg_sh.at[idx_d.at[pl.ds(ch * GS, GS)]],
                         ssems[k], add=True)

    def acc_wait(ch, k):
        pltpu.make_async_copy(bufs[k],
                              agg_sh.at[idx_d.at[pl.ds(ch * GS, GS)]],
                              ssems[k]).wait()

    for k in range(4):
        gat(k, k)

    @pl.loop(0, NCHS - 2, step=4)
    def _(c):
        for k in range(4):
            wat(c + k, k)
            acc(c + k, k)
        for k in range(4):
            acc_wait(c + k, k)

            @pl.when(c + k + 4 < NCHS)
            def _():
                gat(c + k + 4, k)

    for k in range(2):
        wat(NCHS - 2 + k, k)
        acc(NCHS - 2 + k, k)
    for k in range(2):
        acc_wait(NCHS - 2 + k, k)

    plsc.subcore_barrier()

    pltpu.sync_copy(agg_sh.at[pl.ds(sid * RT, RT)],
                    out_hbm.at[cid, pl.ds(sid * RT, RT)])


@functools.partial(
    pl.kernel,
    out_type=jax.ShapeDtypeStruct((2 * E,), jnp.float32),
    mesh=_MESH,
    compiler_params=_SC_PARAMS,
    scratch_types=[
        pltpu.VMEM((2 * ET,), jnp.int32),
        pltpu.VMEM((2 * ET,), jnp.int32),
        pltpu.VMEM((G, D), jnp.float32),
        pltpu.VMEM((G, D), jnp.float32),
        pltpu.VMEM((G, D), jnp.float32),
        pltpu.VMEM((G, D), jnp.float32),
        pltpu.VMEM((G,), jnp.float32),
        pltpu.VMEM((G,), jnp.float32),
        pltpu.SemaphoreType.DMA,
        pltpu.SemaphoreType.DMA,
        pltpu.SemaphoreType.DMA,
        pltpu.SemaphoreType.DMA,
    ],
)
def _sc_scores(h_hbm, psrc, pdst, nsrc, ndst, out,
               idx_s, idx_d, hs0, hd0, hs1, hd1, sc0, sc1,
               gsem0, gsem1, wsem0, wsem1):
    cid = lax.axis_index("c")
    sid = lax.axis_index("s")
    base = cid * EC + sid * ET
    lanes = jnp.arange(16, dtype=jnp.int32)
    _dn = lax.GatherDimensionNumbers(
        offset_dims=(), collapsed_slice_dims=(0,), start_index_map=(0,))

    def shuffle(x, sh):
        return lax.gather(x, ((lanes ^ sh)[:, None]), _dn, slice_sizes=(1,),
                          mode=lax.GatherScatterMode.PROMISE_IN_BOUNDS)

    sbufs = (hs0, hs1)
    dbufs = (hd0, hd1)
    scbufs = (sc0, sc1)
    gsems = (gsem0, gsem1)
    wsems = (wsem0, wsem1)
    NCH2 = 2 * NCH

    pltpu.sync_copy(psrc.at[pl.ds(base, ET)], idx_s.at[pl.ds(0, ET)])
    pltpu.sync_copy(nsrc.at[pl.ds(base, ET)], idx_s.at[pl.ds(ET, ET)])
    pltpu.sync_copy(pdst.at[pl.ds(base, ET)], idx_d.at[pl.ds(0, ET)])
    pltpu.sync_copy(ndst.at[pl.ds(base, ET)], idx_d.at[pl.ds(ET, ET)])

    def off(ch):
        return base + ch * G + jnp.where(ch >= NCH, E - ET, 0)

    def gat(ch, k):
        pltpu.async_copy(h_hbm.at[idx_s.at[pl.ds(ch * G, G)]],
                         sbufs[k], gsems[k])
        pltpu.async_copy(h_hbm.at[idx_d.at[pl.ds(ch * G, G)]],
                         dbufs[k], gsems[k])

    def wat(ch, k):
        pltpu.make_async_copy(h_hbm.at[idx_s.at[pl.ds(ch * G, G)]],
                              sbufs[k], gsems[k]).wait()
        pltpu.make_async_copy(h_hbm.at[idx_d.at[pl.ds(ch * G, G)]],
                              dbufs[k], gsems[k]).wait()

    def put(ch, k):
        pltpu.async_copy(scbufs[k], out.at[pl.ds(off(ch), G)], wsems[k])

    def put_wait(ch, k):
        pltpu.make_async_copy(scbufs[k], out.at[pl.ds(off(ch), G)],
                              wsems[k]).wait()

    def dots(ch, k):
        hs, hd = sbufs[k], dbufs[k]

        @pl.loop(0, G, step=16)
        def _(e0):
            vec = jnp.zeros((16,), jnp.float32)
            for l in range(16):
                e = e0 + l
                acc = hs[e, pl.ds(0, 16)] * hd[e, pl.ds(0, 16)]
                for j in range(1, D // 16):
                    acc = acc + hs[e, pl.ds(j * 16, 16)] * hd[e, pl.ds(j * 16, 16)]
                for sh in (8, 4, 2, 1):
                    acc = acc + shuffle(acc, sh)
                vec = jnp.where(lanes == l, acc, vec)
            scbufs[k][pl.ds(e0, 16)] = vec

    gat(0, 0)
    gat(1, 1)

    @pl.loop(0, NCH2, step=2)
    def _(c):
        for k in range(2):
            wat(c + k, k)

            @pl.when(c >= 2)
            def _():
                put_wait(c + k - 2, k)

            dots(c + k, k)
            put(c + k, k)

            @pl.when(c + k + 2 < NCH2)
            def _():
                gat(c + k + 2, k)

    put_wait(NCH2 - 2, 0)
    put_wait(NCH2 - 1, 1)


def _tc1_body(deg_ref, x_ref, w_ref, out_ref):
    dout = jnp.sum(deg_ref[0], axis=0)
    ns = lax.rsqrt(jnp.maximum(dout, 1.0))[:, None]
    xw = jnp.dot(x_ref[...], w_ref[...], preferred_element_type=jnp.float32)
    out_ref[...] = xw * ns


def _tc_linear1(degp, x, W1):
    return pl.pallas_call(
        _tc1_body,
        grid=(NP // ROWBLK,),
        in_specs=[
            pl.BlockSpec((2, NC * NT, ROWBLK), lambda i: (0, 0, i)),
            pl.BlockSpec((ROWBLK, D), lambda i: (i, 0)),
            pl.BlockSpec((D, D), lambda i: (0, 0)),
        ],
        out_specs=pl.BlockSpec((ROWBLK, D), lambda i: (i, 0)),
        out_shape=jax.ShapeDtypeStruct((NP, D), jnp.float32),
    )(degp, x, W1)


def _tc2_body(deg_ref, s_ref, b_ref, w_ref, out_ref):
    nd = lax.rsqrt(jnp.maximum(jnp.sum(deg_ref[1], axis=0), 1.0))[:, None]
    ns = lax.rsqrt(jnp.maximum(jnp.sum(deg_ref[0], axis=0), 1.0))[:, None]
    h1 = jnp.maximum((s_ref[0] + s_ref[1]) * nd + b_ref[...], 0.0)
    hw = jnp.dot(h1, w_ref[...], preferred_element_type=jnp.float32)
    out_ref[...] = hw * ns


def _tc_mid(degp, s1, b1, W2):
    return pl.pallas_call(
        _tc2_body,
        grid=(NP // ROWBLK,),
        in_specs=[
            pl.BlockSpec((2, NC * NT, ROWBLK), lambda i: (0, 0, i)),
            pl.BlockSpec((NC, ROWBLK, D), lambda i: (0, i, 0)),
            pl.BlockSpec((1, D), lambda i: (0, 0)),
            pl.BlockSpec((D, D), lambda i: (0, 0)),
        ],
        out_specs=pl.BlockSpec((ROWBLK, D), lambda i: (i, 0)),
        out_shape=jax.ShapeDtypeStruct((NP, D), jnp.float32),
    )(degp, s1, b1.reshape(1, D), W2)


def _tc3_body(deg_ref, s_ref, b_ref, out_ref):
    nd = lax.rsqrt(jnp.maximum(jnp.sum(deg_ref[1], axis=0), 1.0))[:, None]
    out_ref[...] = (s_ref[0] + s_ref[1]) * nd + b_ref[...]


def _tc_final(degp, s2, b2):
    return pl.pallas_call(
        _tc3_body,
        grid=(NP // ROWBLK,),
        in_specs=[
            pl.BlockSpec((2, NC * NT, ROWBLK), lambda i: (0, 0, i)),
            pl.BlockSpec((NC, ROWBLK, D), lambda i: (0, i, 0)),
            pl.BlockSpec((1, D), lambda i: (0, 0)),
        ],
        out_specs=pl.BlockSpec((ROWBLK, D), lambda i: (i, 0)),
        out_shape=jax.ShapeDtypeStruct((NP, D), jnp.float32),
    )(degp, s2, b2.reshape(1, D))


def kernel(x, pos_edge_index, neg_edge_index, W1, b1, W2, b2):
    psrc = pos_edge_index[0].astype(jnp.int32)
    pdst = pos_edge_index[1].astype(jnp.int32)
    nsrc = neg_edge_index[0].astype(jnp.int32)
    ndst = neg_edge_index[1].astype(jnp.int32)

    xp = jnp.pad(x, ((0, NP - N), (0, 0)))

    degp = _sc_degrees(psrc, pdst)
    hw1 = _tc_linear1(degp, xp, W1)
    s1 = _sc_segsum(hw1, psrc, pdst)
    hw2 = _tc_mid(degp, s1, b1, W2)
    s2 = _sc_segsum(hw2, psrc, pdst)
    h = _tc_final(degp, s2, b2)
    scores = _sc_scores(h, psrc, pdst, nsrc, ndst)
    return (scores[:E], scores[E:])


# speedup vs baseline: 11.3971x; 1.6052x over previous
import dataclasses
import functools

import jax
import jax.numpy as jnp
from jax import lax
from jax.experimental import pallas as pl
from jax.experimental.pallas import tpu as pltpu
from jax.experimental.pallas import tpu_sc as plsc

N = 10000
NP = 10240
E = 320000
D = 128

NC = 2
NT = 16
EC = E // NC
ET = EC // NT
G = 80
NCH = ET // G
GS = 40
NCHS = ET // GS
RT = NP // NT
ZR = 128

_MESH = plsc.VectorSubcoreMesh(
    core_axis_name="c", subcore_axis_name="s", num_cores=NC, num_subcores=NT
)

_SC_PARAMS = pltpu.CompilerParams()
if "needs_layout_passes" in pltpu.CompilerParams.__dataclass_fields__:
    _SC_PARAMS = dataclasses.replace(_SC_PARAMS, needs_layout_passes=False)
_SC_PARAMS_UNTILED = dataclasses.replace(_SC_PARAMS, use_tc_tiling_on_sc=False)

ROWBLK = 1024


@functools.partial(
    pl.kernel,
    out_type=jax.ShapeDtypeStruct((2, NC * NT, NP), jnp.float32),
    mesh=_MESH,
    compiler_params=_SC_PARAMS,
    scratch_types=[
        pltpu.VMEM((ET,), jnp.int32),
        pltpu.VMEM((NP,), jnp.float32),
        pltpu.VMEM((NP,), jnp.float32),
    ],
)
def _sc_degrees(src_hbm, dst_hbm, out_hbm, idx_v, hsrc_v, hdst_v):
    cid = lax.axis_index("c")
    sid = lax.axis_index("s")
    wid = cid * NT + sid
    base = wid * ET

    @pl.loop(0, NP, step=16)
    def _(i):
        hsrc_v[pl.ds(i, 16)] = jnp.zeros((16,), jnp.float32)
        hdst_v[pl.ds(i, 16)] = jnp.zeros((16,), jnp.float32)

    ones16 = jnp.ones((16,), jnp.float32)

    pltpu.sync_copy(src_hbm.at[pl.ds(base, ET)], idx_v)

    @pl.loop(0, ET, step=16)
    def _(i):
        plsc.addupdate_scatter(hsrc_v, [idx_v[pl.ds(i, 16)]], ones16)

    pltpu.sync_copy(dst_hbm.at[pl.ds(base, ET)], idx_v)

    @pl.loop(0, ET, step=16)
    def _(i):
        plsc.addupdate_scatter(hdst_v, [idx_v[pl.ds(i, 16)]], ones16)

    pltpu.sync_copy(hsrc_v, out_hbm.at[0, wid])
    pltpu.sync_copy(hdst_v, out_hbm.at[1, wid])


@functools.partial(
    pl.kernel,
    out_type=jax.ShapeDtypeStruct((NC, NP, D), jnp.float32),
    mesh=_MESH,
    compiler_params=_SC_PARAMS,
    scratch_types=[
        pltpu.VMEM((ET,), jnp.int32),
        pltpu.VMEM((ET,), jnp.int32),
        pltpu.VMEM((GS, D), jnp.float32),
        pltpu.VMEM((GS, D), jnp.float32),
        pltpu.VMEM((GS, D), jnp.float32),
        pltpu.VMEM((GS, D), jnp.float32),
        pltpu.SemaphoreType.DMA,
        pltpu.SemaphoreType.DMA,
        pltpu.SemaphoreType.DMA,
        pltpu.SemaphoreType.DMA,
        pltpu.SemaphoreType.DMA,
        pltpu.SemaphoreType.DMA,
        pltpu.SemaphoreType.DMA,
        pltpu.SemaphoreType.DMA,
        pltpu.VMEM_SHARED((NP, D), jnp.float32),
    ],
)
def _sc_segsum(hw_hbm, src_hbm, dst_hbm, out_hbm,
               idx_s, idx_d, r0, r1, r2, r3,
               g0, g1, g2, g3, s0, s1, s2, s3, agg_sh):
    cid = lax.axis_index("c")
    sid = lax.axis_index("s")
    bufs = (r0, r1, r2, r3)
    gsems = (g0, g1, g2, g3)
    ssems = (s0, s1, s2, s3)

    @pl.loop(0, GS)
    def _(i):
        @pl.loop(0, D, step=16)
        def _(j):
            r0[i, pl.ds(j, 16)] = jnp.zeros((16,), jnp.float32)

    @pl.loop(0, RT // GS)
    def _(j):
        pltpu.sync_copy(r0, agg_sh.at[pl.ds(sid * RT + j * GS, GS)])

    base = cid * EC + sid * ET
    pltpu.sync_copy(src_hbm.at[pl.ds(base, ET)], idx_s)
    pltpu.sync_copy(dst_hbm.at[pl.ds(base, ET)], idx_d)

    plsc.subcore_barrier()

    def gat(ch, k):
        pltpu.async_copy(hw_hbm.at[idx_s.at[pl.ds(ch * GS, GS)]],
                         bufs[k], gsems[k])

    def wat(ch, k):
        pltpu.make_async_copy(hw_hbm.at[idx_s.at[pl.ds(ch * GS, GS)]],
                              bufs[k], gsems[k]).wait()

    def acc(ch, k):
        pltpu.async_copy(bufs[k], agg_sh.at[idx_d.at[pl.ds(ch * GS, GS)]],
                         ssems[k], add=True)

    def acc_wait(ch, k):
        pltpu.make_async_copy(bufs[k],
                              agg_sh.at[idx_d.at[pl.ds(ch * GS, GS)]],
                              ssems[k]).wait()

    for k in range(4):
        gat(k, k)

    @pl.loop(0, NCHS - 2, step=4)
    def _(c):
        for k in range(4):
            wat(c + k, k)
            acc(c + k, k)
        for k in range(4):
            acc_wait(c + k, k)

            @pl.when(c + k + 4 < NCHS)
            def _():
                gat(c + k + 4, k)

    for k in range(2):
        wat(NCHS - 2 + k, k)
        acc(NCHS - 2 + k, k)
    for k in range(2):
        acc_wait(NCHS - 2 + k, k)

    plsc.subcore_barrier()

    pltpu.sync_copy(agg_sh.at[pl.ds(sid * RT, RT)],
                    out_hbm.at[cid, pl.ds(sid * RT, RT)])


@functools.partial(
    pl.kernel,
    out_type=jax.ShapeDtypeStruct((2 * E,), jnp.float32),
    mesh=_MESH,
    compiler_params=_SC_PARAMS_UNTILED,
    scratch_types=[
        pltpu.VMEM((2 * ET,), jnp.int32),
        pltpu.VMEM((2 * ET,), jnp.int32),
        pltpu.VMEM((G, D // 2), jnp.int32),
        pltpu.VMEM((G, D // 2), jnp.int32),
        pltpu.VMEM((G, D // 2), jnp.int32),
        pltpu.VMEM((G, D // 2), jnp.int32),
        pltpu.VMEM((G,), jnp.float32),
        pltpu.VMEM((G,), jnp.float32),
        pltpu.SemaphoreType.DMA,
        pltpu.SemaphoreType.DMA,
        pltpu.SemaphoreType.DMA,
        pltpu.SemaphoreType.DMA,
    ],
)
def _sc_scores(h_hbm, psrc, pdst, nsrc, ndst, out,
               idx_s, idx_d, hs0, hd0, hs1, hd1, sc0, sc1,
               gsem0, gsem1, wsem0, wsem1):
    cid = lax.axis_index("c")
    sid = lax.axis_index("s")
    base = cid * EC + sid * ET
    lanes = jnp.arange(16, dtype=jnp.int32)
    _dn = lax.GatherDimensionNumbers(
        offset_dims=(), collapsed_slice_dims=(0,), start_index_map=(0,))

    def shuffle(x, sh):
        return lax.gather(x, ((lanes ^ sh)[:, None]), _dn, slice_sizes=(1,),
                          mode=lax.GatherScatterMode.PROMISE_IN_BOUNDS)

    sbufs = (hs0, hs1)
    dbufs = (hd0, hd1)
    scbufs = (sc0, sc1)
    gsems = (gsem0, gsem1)
    wsems = (wsem0, wsem1)
    NCH2 = 2 * NCH

    pltpu.sync_copy(psrc.at[pl.ds(base, ET)], idx_s.at[pl.ds(0, ET)])
    pltpu.sync_copy(nsrc.at[pl.ds(base, ET)], idx_s.at[pl.ds(ET, ET)])
    pltpu.sync_copy(pdst.at[pl.ds(base, ET)], idx_d.at[pl.ds(0, ET)])
    pltpu.sync_copy(ndst.at[pl.ds(base, ET)], idx_d.at[pl.ds(ET, ET)])

    def off(ch):
        return base + ch * G + jnp.where(ch >= NCH, E - ET, 0)

    def gat(ch, k):
        pltpu.async_copy(h_hbm.at[idx_s.at[pl.ds(ch * G, G)]],
                         sbufs[k], gsems[k])
        pltpu.async_copy(h_hbm.at[idx_d.at[pl.ds(ch * G, G)]],
                         dbufs[k], gsems[k])

    def wat(ch, k):
        pltpu.make_async_copy(h_hbm.at[idx_s.at[pl.ds(ch * G, G)]],
                              sbufs[k], gsems[k]).wait()
        pltpu.make_async_copy(h_hbm.at[idx_d.at[pl.ds(ch * G, G)]],
                              dbufs[k], gsems[k]).wait()

    def put(ch, k):
        pltpu.async_copy(scbufs[k], out.at[pl.ds(off(ch), G)], wsems[k])

    def put_wait(ch, k):
        pltpu.make_async_copy(scbufs[k], out.at[pl.ds(off(ch), G)],
                              wsems[k]).wait()

    def dots(ch, k):
        hs, hd = sbufs[k], dbufs[k]

        @pl.loop(0, G, step=16)
        def _(e0):
            vec = jnp.zeros((16,), jnp.float32)
            for l in range(16):
                e = e0 + l
                acc = None
                for j in range(D // 32):
                    sw = plsc.bitcast(hs[e, pl.ds(j * 16, 16)], jnp.bfloat16)
                    dw = plsc.bitcast(hd[e, pl.ds(j * 16, 16)], jnp.bfloat16)
                    sa, sb = plsc.unpack(sw, format=plsc.PackFormat.INTERLEAVED)
                    da, db = plsc.unpack(dw, format=plsc.PackFormat.INTERLEAVED)
                    t = sa * da + sb * db
                    acc = t if acc is None else acc + t
                for sh in (8, 4, 2, 1):
                    acc = acc + shuffle(acc, sh)
                vec = jnp.where(lanes == l, acc, vec)
            scbufs[k][pl.ds(e0, 16)] = vec

    gat(0, 0)
    gat(1, 1)

    @pl.loop(0, NCH2, step=2)
    def _(c):
        for k in range(2):
            wat(c + k, k)

            @pl.when(c >= 2)
            def _():
                put_wait(c + k - 2, k)

            dots(c + k, k)
            put(c + k, k)

            @pl.when(c + k + 2 < NCH2)
            def _():
                gat(c + k + 2, k)

    put_wait(NCH2 - 2, 0)
    put_wait(NCH2 - 1, 1)


def _tc1_body(deg_ref, x_ref, w_ref, out_ref):
    dout = jnp.sum(deg_ref[0], axis=0)
    ns = lax.rsqrt(jnp.maximum(dout, 1.0))[:, None]
    xw = jnp.dot(x_ref[...], w_ref[...], preferred_element_type=jnp.float32)
    out_ref[...] = xw * ns


def _tc_linear1(degp, x, W1):
    return pl.pallas_call(
        _tc1_body,
        grid=(NP // ROWBLK,),
        in_specs=[
            pl.BlockSpec((2, NC * NT, ROWBLK), lambda i: (0, 0, i)),
            pl.BlockSpec((ROWBLK, D), lambda i: (i, 0)),
            pl.BlockSpec((D, D), lambda i: (0, 0)),
        ],
        out_specs=pl.BlockSpec((ROWBLK, D), lambda i: (i, 0)),
        out_shape=jax.ShapeDtypeStruct((NP, D), jnp.float32),
    )(degp, x, W1)


def _tc2_body(deg_ref, s_ref, b_ref, w_ref, out_ref):
    nd = lax.rsqrt(jnp.maximum(jnp.sum(deg_ref[1], axis=0), 1.0))[:, None]
    ns = lax.rsqrt(jnp.maximum(jnp.sum(deg_ref[0], axis=0), 1.0))[:, None]
    h1 = jnp.maximum((s_ref[0] + s_ref[1]) * nd + b_ref[...], 0.0)
    hw = jnp.dot(h1, w_ref[...], preferred_element_type=jnp.float32)
    out_ref[...] = hw * ns


def _tc_mid(degp, s1, b1, W2):
    return pl.pallas_call(
        _tc2_body,
        grid=(NP // ROWBLK,),
        in_specs=[
            pl.BlockSpec((2, NC * NT, ROWBLK), lambda i: (0, 0, i)),
            pl.BlockSpec((NC, ROWBLK, D), lambda i: (0, i, 0)),
            pl.BlockSpec((1, D), lambda i: (0, 0)),
            pl.BlockSpec((D, D), lambda i: (0, 0)),
        ],
        out_specs=pl.BlockSpec((ROWBLK, D), lambda i: (i, 0)),
        out_shape=jax.ShapeDtypeStruct((NP, D), jnp.float32),
    )(degp, s1, b1.reshape(1, D), W2)


def _tc3_body(deg_ref, s_ref, b_ref, out_ref):
    nd = lax.rsqrt(jnp.maximum(jnp.sum(deg_ref[1], axis=0), 1.0))[:, None]
    out_ref[...] = ((s_ref[0] + s_ref[1]) * nd + b_ref[...]).astype(jnp.bfloat16)


def _tc_final(degp, s2, b2):
    return pl.pallas_call(
        _tc3_body,
        grid=(NP // ROWBLK,),
        in_specs=[
            pl.BlockSpec((2, NC * NT, ROWBLK), lambda i: (0, 0, i)),
            pl.BlockSpec((NC, ROWBLK, D), lambda i: (0, i, 0)),
            pl.BlockSpec((1, D), lambda i: (0, 0)),
        ],
        out_specs=pl.BlockSpec((ROWBLK, D), lambda i: (i, 0)),
        out_shape=jax.ShapeDtypeStruct((NP, D), jnp.bfloat16),
    )(degp, s2, b2.reshape(1, D))


def kernel(x, pos_edge_index, neg_edge_index, W1, b1, W2, b2):
    psrc = pos_edge_index[0].astype(jnp.int32)
    pdst = pos_edge_index[1].astype(jnp.int32)
    nsrc = neg_edge_index[0].astype(jnp.int32)
    ndst = neg_edge_index[1].astype(jnp.int32)

    xp = jnp.pad(x, ((0, NP - N), (0, 0)))

    degp = _sc_degrees(psrc, pdst)
    hw1 = _tc_linear1(degp, xp, W1)
    s1 = _sc_segsum(hw1, psrc, pdst)
    hw2 = _tc_mid(degp, s1, b1, W2)
    s2 = _sc_segsum(hw2, psrc, pdst)
    h = _tc_final(degp, s2, b2)
    h32 = lax.bitcast_convert_type(h.reshape(NP, D // 2, 2), jnp.int32)
    scores = _sc_scores(h32, psrc, pdst, nsrc, ndst)
    return (scores[:E], scores[E:])
